# Initial kernel scaffold; baseline (speedup 1.0000x reference)
import jax
import jax.numpy as jnp
from jax.experimental import pallas as pl


def kernel(x, edge_index, W1l, W1r, b1, W2l, W2r, b2, Wlin, blin):
    raise NotImplementedError("write your pallas kernel here")


# speedup vs baseline: 7.4361x; 7.4361x over previous
import functools

import jax
import jax.numpy as jnp
from jax import lax
from jax.experimental import pallas as pl
from jax.experimental.pallas import tpu as pltpu
from jax.experimental.pallas import tpu_sc as plsc

N = 10000
D = 128
H = D // 2
E = 320000
NC = 2
NS = 16
NPAD = 10240
RPT = NPAD // NS
EPT = E // NS
K = 400
NIT = EPT // K

_mesh = plsc.VectorSubcoreMesh(core_axis_name="c", subcore_axis_name="s")


@functools.partial(
    pl.kernel,
    out_type=(
        jax.ShapeDtypeStruct((NC, NPAD, H), jnp.float32),
        jax.ShapeDtypeStruct((NPAD,), jnp.float32),
    ),
    mesh=_mesh,
    compiler_params=pltpu.CompilerParams(use_tc_tiling_on_sc=False),
    scratch_types=[
        pltpu.VMEM((K,), jnp.int32),
        pltpu.VMEM((K,), jnp.int32),
        pltpu.VMEM((K, H), jnp.float32),
        pltpu.VMEM((K,), jnp.float32),
        pltpu.VMEM_SHARED((NPAD, H), jnp.float32),
        pltpu.VMEM_SHARED((NPAD,), jnp.float32),
        pltpu.SemaphoreType.DMA,
    ],
)
def _sc_aggregate(x_hbm, src_hbm, dst_hbm, zrow_hbm, zdeg_hbm, ones_hbm,
                  acc_hbm, deg_hbm,
                  srcb, dstb, rows, ones_v, acc_s, deg_s, sem):
    c = lax.axis_index("c")
    s = lax.axis_index("s")

    pltpu.sync_copy(zrow_hbm.at[pl.ds(s * RPT, RPT)], acc_s.at[pl.ds(s * RPT, RPT)])

    @pl.when((s == 0) & (c == 0))
    def _():
        pltpu.sync_copy(zdeg_hbm, deg_s)

    pltpu.sync_copy(ones_hbm, ones_v)
    plsc.subcore_barrier()

    e0 = s * EPT

    def body(i, carry):
        base = e0 + i * K
        pltpu.sync_copy(src_hbm.at[pl.ds(base, K)], srcb)
        pltpu.sync_copy(dst_hbm.at[pl.ds(base, K)], dstb)
        pltpu.async_copy(x_hbm.at[c].at[srcb], rows, sem).wait()
        pltpu.sync_copy(rows, acc_s.at[dstb], add=True)

        @pl.when(c == 0)
        def _():
            pltpu.sync_copy(ones_v, deg_s.at[dstb], add=True)

        return carry

    lax.fori_loop(0, NIT, body, 0)
    plsc.subcore_barrier()

    pltpu.sync_copy(acc_s.at[pl.ds(s * RPT, RPT)], acc_hbm.at[c, pl.ds(s * RPT, RPT)])

    @pl.when((s == 0) & (c == 0))
    def _():
        pltpu.sync_copy(deg_s, deg_hbm)


RB = 2048


def _sage_block(acc_ref, degb_ref, xh_ref, wl_ref, wr_ref, b_ref):
    d = jnp.maximum(degb_ref[:, :H], 1.0)
    mean_l = acc_ref[0] / d
    mean_r = acc_ref[1] / d
    h = (
        jnp.dot(mean_l, wl_ref[:H, :], preferred_element_type=jnp.float32)
        + jnp.dot(mean_r, wl_ref[H:, :], preferred_element_type=jnp.float32)
        + jnp.dot(xh_ref[0], wr_ref[:H, :], preferred_element_type=jnp.float32)
        + jnp.dot(xh_ref[1], wr_ref[H:, :], preferred_element_type=jnp.float32)
        + b_ref[...]
    )
    return jnp.maximum(h, 0.0)


def _dense_body(acc_ref, degb_ref, xh_ref, wl_ref, wr_ref, b_ref, out_ref):
    h = _sage_block(acc_ref, degb_ref, xh_ref, wl_ref, wr_ref, b_ref)
    out_ref[0] = h[:, :H]
    out_ref[1] = h[:, H:]


def _dense_final_body(acc_ref, degb_ref, xh_ref, wl_ref, wr_ref, b_ref,
                      wlin_ref, blin_ref, out_ref):
    h = _sage_block(acc_ref, degb_ref, xh_ref, wl_ref, wr_ref, b_ref)
    out_ref[...] = (
        jnp.dot(h, wlin_ref[...], preferred_element_type=jnp.float32)
        + blin_ref[...]
    )


_half_spec = pl.BlockSpec((NC, RB, H), lambda i: (0, i, 0))
_row_spec = pl.BlockSpec((RB, D), lambda i: (i, 0))
_w_spec = pl.BlockSpec((D, D), lambda i: (0, 0))
_b_spec = pl.BlockSpec((1, D), lambda i: (0, 0))

_dense1 = pl.pallas_call(
    _dense_body,
    grid=(NPAD // RB,),
    in_specs=[_half_spec, _row_spec, _half_spec, _w_spec, _w_spec, _b_spec],
    out_specs=_half_spec,
    out_shape=jax.ShapeDtypeStruct((NC, NPAD, H), jnp.float32),
)

_dense2 = pl.pallas_call(
    _dense_final_body,
    grid=(NPAD // RB,),
    in_specs=[_half_spec, _row_spec, _half_spec, _w_spec, _w_spec, _b_spec,
              _w_spec, _b_spec],
    out_specs=_row_spec,
    out_shape=jax.ShapeDtypeStruct((NPAD, D), jnp.float32),
)


def kernel(x, edge_index, W1l, W1r, b1, W2l, W2r, b2, Wlin, blin):
    x = x.astype(jnp.float32)
    src = edge_index[0].astype(jnp.int32)
    dst = edge_index[1].astype(jnp.int32)
    zrow = jnp.zeros((NPAD, H), jnp.float32)
    zdeg = jnp.zeros((NPAD,), jnp.float32)
    ones = jnp.ones((K,), jnp.float32)
    xp = jnp.pad(x, ((0, NPAD - N), (0, 0)))
    xh = jnp.stack([xp[:, :H], xp[:, H:]])

    acc1, deg = _sc_aggregate(xh, src, dst, zrow, zdeg, ones)
    degb = jnp.broadcast_to(deg[:, None], (NPAD, D))
    h1 = _dense1(acc1, degb, xh, W1l, W1r, b1.reshape(1, D))

    acc2, _ = _sc_aggregate(h1, src, dst, zrow, zdeg, ones)
    out = _dense2(acc2, degb, h1, W2l, W2r, b2.reshape(1, D),
                  Wlin, blin.reshape(1, D))
    return out[:N]


# speedup vs baseline: 12.5162x; 1.6832x over previous
import functools

import jax
import jax.numpy as jnp
from jax import lax
from jax.experimental import pallas as pl
from jax.experimental.pallas import tpu as pltpu
from jax.experimental.pallas import tpu_sc as plsc

N = 10000
D = 128
H = D // 2
E = 320000
NC = 2
NS = 16
NPAD = 10240
RPT = NPAD // NS
EPT = E // NS
K = 200
NIT = EPT // K
NBUF = 3
P = next(p for p in range(2, 2 + NBUF) if (NIT - p - 3) % NBUF == 0)
NGROUPS = (NIT - P - 3) // NBUF

_mesh = plsc.VectorSubcoreMesh(core_axis_name="c", subcore_axis_name="s")


def _make_sc_aggregate(compute_deg):
    out_type = [jax.ShapeDtypeStruct((NC, NPAD, H), jnp.float32)]
    if compute_deg:
        out_type.append(jax.ShapeDtypeStruct((NPAD,), jnp.float32))

    scratch = [
        pltpu.VMEM((NIT, K), jnp.int32),
        pltpu.VMEM((NIT, K), jnp.int32),
        pltpu.VMEM((NBUF, K, H), jnp.float32),
        pltpu.VMEM((K,), jnp.float32),
        pltpu.VMEM_SHARED((NPAD, H), jnp.float32),
        pltpu.VMEM_SHARED((NPAD,), jnp.float32),
    ] + [pltpu.SemaphoreType.DMA] * (3 * NBUF)

    def body(x_hbm, src_hbm, dst_hbm, zrow_hbm, zdeg_hbm, ones_hbm,
             *refs):
        if compute_deg:
            acc_hbm, deg_hbm = refs[0], refs[1]
            rest = refs[2:]
        else:
            acc_hbm = refs[0]
            rest = refs[1:]
        (srcb, dstb, rows, ones_v, acc_s, deg_s) = rest[:6]
        sem_g = rest[6:6 + NBUF]
        sem_s = rest[6 + NBUF:6 + 2 * NBUF]
        sem_d = rest[6 + 2 * NBUF:6 + 3 * NBUF]

        c = lax.axis_index("c")
        s = lax.axis_index("s")

        pltpu.sync_copy(src_hbm.at[s], srcb)
        pltpu.sync_copy(dst_hbm.at[s], dstb)
        pltpu.sync_copy(zrow_hbm.at[pl.ds(s * RPT, RPT)],
                        acc_s.at[pl.ds(s * RPT, RPT)])
        if compute_deg:
            pltpu.sync_copy(ones_hbm, ones_v)

            @pl.when((s == 0) & (c == 0))
            def _():
                pltpu.sync_copy(zdeg_hbm, deg_s)

        plsc.subcore_barrier()

        def start_gather(i, b):
            pltpu.async_copy(x_hbm.at[c].at[srcb.at[i]], rows.at[b], sem_g[b])

        def wait_gather(i, b):
            pltpu.make_async_copy(x_hbm.at[c].at[srcb.at[i]], rows.at[b],
                                  sem_g[b]).wait()

        def start_scatter(i, b):
            pltpu.async_copy(rows.at[b], acc_s.at[dstb.at[i]], sem_s[b],
                             add=True)
            if compute_deg:
                @pl.when(c == 0)
                def _():
                    pltpu.async_copy(ones_v, deg_s.at[dstb.at[i]], sem_d[b],
                                     add=True)

        def wait_scatter(i, b):
            pltpu.make_async_copy(rows.at[b], acc_s.at[dstb.at[i]],
                                  sem_s[b]).wait()
            if compute_deg:
                @pl.when(c == 0)
                def _():
                    pltpu.make_async_copy(ones_v, deg_s.at[dstb.at[i]],
                                          sem_d[b]).wait()

        def emit_iter(i, b, b2, wait_prev=True, emit_next=True):
            wait_gather(i, b)
            start_scatter(i, b)
            if emit_next:
                if wait_prev:
                    wait_scatter(i - 1, b2)
                start_gather(i + 2, b2)

        start_gather(0, 0)
        start_gather(1, 1 % NBUF)
        for i in range(P):
            emit_iter(i, i % NBUF, (i + 2) % NBUF, wait_prev=(i >= 1))

        def group(g, carry):
            i0 = P + g * NBUF
            for u in range(NBUF):
                emit_iter(i0 + u, (P + u) % NBUF, (P + u + 2) % NBUF)
            return carry

        lax.fori_loop(0, NGROUPS, group, 0)

        emit_iter(NIT - 3, (NIT - 3) % NBUF, (NIT - 1) % NBUF)
        emit_iter(NIT - 2, (NIT - 2) % NBUF, 0, emit_next=False)
        emit_iter(NIT - 1, (NIT - 1) % NBUF, 0, emit_next=False)
        wait_scatter(NIT - 3, (NIT - 3) % NBUF)
        wait_scatter(NIT - 2, (NIT - 2) % NBUF)
        wait_scatter(NIT - 1, (NIT - 1) % NBUF)

        plsc.subcore_barrier()

        pltpu.sync_copy(acc_s.at[pl.ds(s * RPT, RPT)],
                        acc_hbm.at[c, pl.ds(s * RPT, RPT)])
        if compute_deg:
            @pl.when((s == 0) & (c == 0))
            def _():
                pltpu.sync_copy(deg_s, deg_hbm)

    return pl.kernel(
        body,
        out_type=tuple(out_type) if compute_deg else out_type[0],
        mesh=_mesh,
        compiler_params=pltpu.CompilerParams(use_tc_tiling_on_sc=False),
        scratch_types=scratch,
    )


_sc_aggregate_deg = _make_sc_aggregate(True)
_sc_aggregate = _make_sc_aggregate(False)


RB = 2048


def _sage_block(acc_ref, degb_ref, xh_ref, wl_ref, wr_ref, b_ref):
    d = jnp.maximum(degb_ref[:, :H], 1.0)
    mean_l = acc_ref[0] / d
    mean_r = acc_ref[1] / d
    h = (
        jnp.dot(mean_l, wl_ref[:H, :], preferred_element_type=jnp.float32)
        + jnp.dot(mean_r, wl_ref[H:, :], preferred_element_type=jnp.float32)
        + jnp.dot(xh_ref[0], wr_ref[:H, :], preferred_element_type=jnp.float32)
        + jnp.dot(xh_ref[1], wr_ref[H:, :], preferred_element_type=jnp.float32)
        + b_ref[...]
    )
    return jnp.maximum(h, 0.0)


def _dense_body(acc_ref, degb_ref, xh_ref, wl_ref, wr_ref, b_ref, out_ref):
    h = _sage_block(acc_ref, degb_ref, xh_ref, wl_ref, wr_ref, b_ref)
    out_ref[0] = h[:, :H]
    out_ref[1] = h[:, H:]


def _dense_final_body(acc_ref, degb_ref, xh_ref, wl_ref, wr_ref, b_ref,
                      wlin_ref, blin_ref, out_ref):
    h = _sage_block(acc_ref, degb_ref, xh_ref, wl_ref, wr_ref, b_ref)
    out_ref[...] = (
        jnp.dot(h, wlin_ref[...], preferred_element_type=jnp.float32)
        + blin_ref[...]
    )


_half_spec = pl.BlockSpec((NC, RB, H), lambda i: (0, i, 0))
_row_spec = pl.BlockSpec((RB, D), lambda i: (i, 0))
_w_spec = pl.BlockSpec((D, D), lambda i: (0, 0))
_b_spec = pl.BlockSpec((1, D), lambda i: (0, 0))

_dense1 = pl.pallas_call(
    _dense_body,
    grid=(NPAD // RB,),
    in_specs=[_half_spec, _row_spec, _half_spec, _w_spec, _w_spec, _b_spec],
    out_specs=_half_spec,
    out_shape=jax.ShapeDtypeStruct((NC, NPAD, H), jnp.float32),
)

_dense2 = pl.pallas_call(
    _dense_final_body,
    grid=(NPAD // RB,),
    in_specs=[_half_spec, _row_spec, _half_spec, _w_spec, _w_spec, _b_spec,
              _w_spec, _b_spec],
    out_specs=_row_spec,
    out_shape=jax.ShapeDtypeStruct((NPAD, D), jnp.float32),
)


def kernel(x, edge_index, W1l, W1r, b1, W2l, W2r, b2, Wlin, blin):
    x = x.astype(jnp.float32)
    src = edge_index[0].astype(jnp.int32).reshape(NS, NIT, K)
    dst = edge_index[1].astype(jnp.int32).reshape(NS, NIT, K)
    zrow = jnp.zeros((NPAD, H), jnp.float32)
    zdeg = jnp.zeros((NPAD,), jnp.float32)
    ones = jnp.ones((K,), jnp.float32)
    xp = jnp.pad(x, ((0, NPAD - N), (0, 0)))
    xh = jnp.stack([xp[:, :H], xp[:, H:]])

    acc1, deg = _sc_aggregate_deg(xh, src, dst, zrow, zdeg, ones)
    degb = jnp.broadcast_to(deg[:, None], (NPAD, D))
    h1 = _dense1(acc1, degb, xh, W1l, W1r, b1.reshape(1, D))

    acc2 = _sc_aggregate(h1, src, dst, zrow, zdeg, ones)
    out = _dense2(acc2, degb, h1, W2l, W2r, b2.reshape(1, D),
                  Wlin, blin.reshape(1, D))
    return out[:N]
